# Initial kernel scaffold; baseline (speedup 1.0000x reference)
import functools

import jax
import jax.numpy as jnp
from jax import lax
from jax.experimental import pallas as pl
from jax.experimental.pallas import tpu as pltpu
from jax.experimental.pallas import tpu_sc as plsc

N = 10000
E = 320000
D = 128
DE = 16

NC = 2
NS = 16
NW = NC * NS

EW = E // NW
C = 80
NCH = EW // C
NR_TILE = N // NS
ZR = 125


def _edge_body(src_hbm, dst_hbm, k_hbm, qv_hbm, e_hbm, out_hbm,
               srcv, dstv, kv, qvv, ev, msgv, zbuf, aggr_sh, sem0, sem1):
    c = lax.axis_index("c")
    s = lax.axis_index("s")
    w = c * NS + s

    def zrow(i, _):
        for db in range(D // 16):
            zbuf[i, pl.ds(db * 16, 16)] = jnp.zeros((16,), jnp.float32)
        return 0
    lax.fori_loop(0, ZR, zrow, 0)
    row0 = s * NR_TILE
    for j in range(NR_TILE // ZR):
        pltpu.sync_copy(zbuf, aggr_sh.at[pl.ds(row0 + j * ZR, ZR)])
    plsc.subcore_barrier()

    base0 = w * EW

    def chunk(ci, _):
        base = base0 + ci * C
        pltpu.sync_copy(src_hbm.at[pl.ds(base, C)], srcv)
        pltpu.sync_copy(dst_hbm.at[pl.ds(base, C)], dstv)
        cp_k = pltpu.async_copy(k_hbm.at[dstv], kv, sem0)
        cp_qv = pltpu.async_copy(qv_hbm.at[srcv], qvv, sem1)
        pltpu.sync_copy(e_hbm.at[pl.ds(base, C)], ev)
        cp_k.wait()
        cp_qv.wait()

        def row(i, _):
            for db in range(D // 16):
                sl = pl.ds(db * 16, 16)
                z = kv[i, sl] + qvv[i, sl] + ev[i, sl]
                g = 1.0 / (1.0 + jnp.exp(-z))
                msgv[i, sl] = g * qvv[i, pl.ds(D + db * 16, 16)]
            return 0
        lax.fori_loop(0, C, row, 0)

        pltpu.sync_copy(msgv, aggr_sh.at[dstv], add=True)
        return 0

    lax.fori_loop(0, NCH, chunk, 0)
    plsc.subcore_barrier()

    for j in range(NR_TILE // ZR):
        r = row0 + j * ZR
        pltpu.sync_copy(aggr_sh.at[pl.ds(r, ZR)], zbuf)
        pltpu.sync_copy(zbuf, out_hbm.at[c, pl.ds(r, ZR)])


@functools.lru_cache(maxsize=None)
def _edge_kernel():
    mesh = plsc.VectorSubcoreMesh(
        core_axis_name="c", subcore_axis_name="s",
        num_cores=NC, num_subcores=NS)
    return functools.partial(
        pl.kernel,
        out_type=jax.ShapeDtypeStruct((NC, N, D), jnp.float32),
        mesh=mesh,
        scratch_types=[
            pltpu.VMEM((C,), jnp.int32),
            pltpu.VMEM((C,), jnp.int32),
            pltpu.VMEM((C, D), jnp.float32),
            pltpu.VMEM((C, 2 * D), jnp.float32),
            pltpu.VMEM((C, D), jnp.float32),
            pltpu.VMEM((C, D), jnp.float32),
            pltpu.VMEM((ZR, D), jnp.float32),
            pltpu.VMEM_SHARED((N, D), jnp.float32),
            pltpu.SemaphoreType.DMA,
            pltpu.SemaphoreType.DMA,
        ],
    )(_edge_body)


RN = 2000
GN = N // RN
RE = 8000
GE = E // RE


def _dense_b(x_ref, wk, bk, wqv, bqv, ws, bs, k_ref, qv_ref, s_ref):
    xb = x_ref[...]
    k_ref[...] = jnp.dot(xb, wk[...], preferred_element_type=jnp.float32) + bk[...]
    qv_ref[...] = jnp.dot(xb, wqv[...], preferred_element_type=jnp.float32) + bqv[...]
    s_ref[...] = jnp.dot(xb, ws[...], preferred_element_type=jnp.float32) + bs[...]


@functools.lru_cache(maxsize=None)
def _dense_kernel():
    full = lambda shape: pl.BlockSpec(shape, lambda i: (0, 0))
    return pl.pallas_call(
        _dense_b,
        grid=(GN,),
        in_specs=[
            pl.BlockSpec((RN, D), lambda i: (i, 0)),
            full((D, D)), full((1, D)),
            full((D, 2 * D)), full((1, 2 * D)),
            full((D, D)), full((1, D)),
        ],
        out_specs=[
            pl.BlockSpec((RN, D), lambda i: (i, 0)),
            pl.BlockSpec((RN, 2 * D), lambda i: (i, 0)),
            pl.BlockSpec((RN, D), lambda i: (i, 0)),
        ],
        out_shape=[
            jax.ShapeDtypeStruct((N, D), jnp.float32),
            jax.ShapeDtypeStruct((N, 2 * D), jnp.float32),
            jax.ShapeDtypeStruct((N, D), jnp.float32),
        ],
    )


def _edgeproj_b(xe_ref, w1, b1, w2, b2, e1_ref, e2_ref):
    xe = xe_ref[...]
    e1_ref[...] = jnp.dot(xe, w1[...], preferred_element_type=jnp.float32) + b1[...]
    e2_ref[...] = jnp.dot(xe, w2[...], preferred_element_type=jnp.float32) + b2[...]


@functools.lru_cache(maxsize=None)
def _edgeproj_kernel():
    full = lambda shape: pl.BlockSpec(shape, lambda i: (0, 0))
    return pl.pallas_call(
        _edgeproj_b,
        grid=(GE,),
        in_specs=[
            pl.BlockSpec((RE, DE), lambda i: (i, 0)),
            full((DE, D)), full((1, D)),
            full((DE, D)), full((1, D)),
        ],
        out_specs=[
            pl.BlockSpec((RE, D), lambda i: (i, 0)),
            pl.BlockSpec((RE, D), lambda i: (i, 0)),
        ],
        out_shape=[
            jax.ShapeDtypeStruct((E, D), jnp.float32),
            jax.ShapeDtypeStruct((E, D), jnp.float32),
        ],
    )


def _combine_b(aggr_ref, s_ref, b_ref, pre_ref, sum_ref, ssq_ref):
    i = pl.program_id(0)
    blk = aggr_ref[0] + aggr_ref[1] + s_ref[...] + b_ref[...]
    pre_ref[...] = blk

    @pl.when(i == 0)
    def _():
        sum_ref[...] = jnp.zeros_like(sum_ref)
        ssq_ref[...] = jnp.zeros_like(ssq_ref)

    sum_ref[...] += jnp.sum(blk, axis=0, keepdims=True)
    ssq_ref[...] += jnp.sum(blk * blk, axis=0, keepdims=True)


@functools.lru_cache(maxsize=None)
def _combine_kernel():
    full = lambda shape: pl.BlockSpec(shape, lambda i: tuple(0 for _ in shape))
    return pl.pallas_call(
        _combine_b,
        grid=(GN,),
        in_specs=[
            pl.BlockSpec((NC, RN, D), lambda i: (0, i, 0)),
            pl.BlockSpec((RN, D), lambda i: (i, 0)),
            full((1, D)),
        ],
        out_specs=[
            pl.BlockSpec((RN, D), lambda i: (i, 0)),
            full((1, D)),
            full((1, D)),
        ],
        out_shape=[
            jax.ShapeDtypeStruct((N, D), jnp.float32),
            jax.ShapeDtypeStruct((1, D), jnp.float32),
            jax.ShapeDtypeStruct((1, D), jnp.float32),
        ],
    )


def _bn_h(pre_ref, sum_ref, ssq_ref, g_ref, bb_ref):
    mu = sum_ref[...] / N
    var = ssq_ref[...] / N - mu * mu
    xn = (pre_ref[...] - mu) * lax.rsqrt(var + 1e-5)
    return jnp.maximum(xn * g_ref[...] + bb_ref[...], 0.0)


def _bn_dense_b(pre_ref, sum_ref, ssq_ref, g_ref, bb_ref,
                wk, bk, wqv, bqv, ws, bs, k_ref, qv_ref, s_ref):
    h = _bn_h(pre_ref, sum_ref, ssq_ref, g_ref, bb_ref)
    k_ref[...] = jnp.dot(h, wk[...], preferred_element_type=jnp.float32) + bk[...]
    qv_ref[...] = jnp.dot(h, wqv[...], preferred_element_type=jnp.float32) + bqv[...]
    s_ref[...] = jnp.dot(h, ws[...], preferred_element_type=jnp.float32) + bs[...]


@functools.lru_cache(maxsize=None)
def _bn_dense_kernel():
    full = lambda shape: pl.BlockSpec(shape, lambda i: (0, 0))
    return pl.pallas_call(
        _bn_dense_b,
        grid=(GN,),
        in_specs=[
            pl.BlockSpec((RN, D), lambda i: (i, 0)),
            full((1, D)), full((1, D)), full((1, D)), full((1, D)),
            full((D, D)), full((1, D)),
            full((D, 2 * D)), full((1, 2 * D)),
            full((D, D)), full((1, D)),
        ],
        out_specs=[
            pl.BlockSpec((RN, D), lambda i: (i, 0)),
            pl.BlockSpec((RN, 2 * D), lambda i: (i, 0)),
            pl.BlockSpec((RN, D), lambda i: (i, 0)),
        ],
        out_shape=[
            jax.ShapeDtypeStruct((N, D), jnp.float32),
            jax.ShapeDtypeStruct((N, 2 * D), jnp.float32),
            jax.ShapeDtypeStruct((N, D), jnp.float32),
        ],
    )


def _bn_apply_b(pre_ref, sum_ref, ssq_ref, g_ref, bb_ref, out_ref):
    out_ref[...] = _bn_h(pre_ref, sum_ref, ssq_ref, g_ref, bb_ref)


@functools.lru_cache(maxsize=None)
def _bn_apply_kernel():
    full = lambda shape: pl.BlockSpec(shape, lambda i: (0, 0))
    return pl.pallas_call(
        _bn_apply_b,
        grid=(GN,),
        in_specs=[
            pl.BlockSpec((RN, D), lambda i: (i, 0)),
            full((1, D)), full((1, D)), full((1, D)), full((1, D)),
        ],
        out_specs=pl.BlockSpec((RN, D), lambda i: (i, 0)),
        out_shape=jax.ShapeDtypeStruct((N, D), jnp.float32),
    )


def kernel(x, edge_index, x_edge,
           Wk1, bk1, Wq1, bq1, Wv1, bv1, We1, be1, Ws1, b1, bng1, bnb1,
           Wk2, bk2, Wq2, bq2, Wv2, bv2, We2, be2, Ws2, b2, bng2, bnb2):
    src = edge_index[0]
    dst = edge_index[1]

    r1 = lambda v: v.reshape(1, D)
    wqv1 = jnp.concatenate([Wq1.T, Wv1.T], axis=1)
    bqv1 = jnp.concatenate([bq1, bv1]).reshape(1, 2 * D)
    wqv2 = jnp.concatenate([Wq2.T, Wv2.T], axis=1)
    bqv2 = jnp.concatenate([bq2, bv2]).reshape(1, 2 * D)

    e1, e2 = _edgeproj_kernel()(x_edge, We1.T, r1(be1), We2.T, r1(be2))

    k1, qv1, s1 = _dense_kernel()(
        x, Wk1.T, r1(bk1), wqv1, bqv1, Ws1.T, r1(b1))
    aggr1 = _edge_kernel()(src, dst, k1, qv1, e1)
    pre1, sum1, ssq1 = _combine_kernel()(aggr1, s1, jnp.zeros((1, D), jnp.float32))

    k2, qv2, s2 = _bn_dense_kernel()(
        pre1, sum1, ssq1, r1(bng1), r1(bnb1),
        Wk2.T, r1(bk2), wqv2, bqv2, Ws2.T, r1(b2))
    aggr2 = _edge_kernel()(src, dst, k2, qv2, e2)
    pre2, sum2, ssq2 = _combine_kernel()(aggr2, s2, jnp.zeros((1, D), jnp.float32))

    return _bn_apply_kernel()(pre2, sum2, ssq2, r1(bng2), r1(bnb2))


# speedup vs baseline: 1.3101x; 1.3101x over previous
import functools

import jax
import jax.numpy as jnp
from jax import lax
from jax.experimental import pallas as pl
from jax.experimental.pallas import tpu as pltpu
from jax.experimental.pallas import tpu_sc as plsc

N = 10000
E = 320000
D = 128
DE = 16

NC = 2
NS = 16
NW = NC * NS

EW = E // NW
C = 80
NCH = EW // C
NPAD = 10240
NR_TILE = NPAD // NS
ZR = C


def _edge_body(src_hbm, dst_hbm, k_hbm, qv_hbm, e_hbm, out_hbm,
               srcv, dstv, kv, qvv, ev, aggr_sh, sem0, sem1):
    c = lax.axis_index("c")
    s = lax.axis_index("s")
    w = c * NS + s

    def zrow(i, _):
        for db in range(D // 16):
            kv[i, pl.ds(db * 16, 16)] = jnp.zeros((16,), jnp.float32)
        return 0
    lax.fori_loop(0, ZR, zrow, 0)
    row0 = s * NR_TILE
    for j in range(NR_TILE // ZR):
        pltpu.sync_copy(kv, aggr_sh.at[pl.ds(row0 + j * ZR, ZR)])
    plsc.subcore_barrier()

    base0 = w * EW

    def chunk(ci, _):
        base = base0 + ci * C
        pltpu.sync_copy(src_hbm.at[pl.ds(base, C)], srcv)
        pltpu.sync_copy(dst_hbm.at[pl.ds(base, C)], dstv)
        cp_k = pltpu.async_copy(k_hbm.at[dstv], kv, sem0)
        cp_qv = pltpu.async_copy(qv_hbm.at[srcv], qvv, sem1)
        pltpu.sync_copy(e_hbm.at[pl.ds(base, C)], ev)
        cp_k.wait()
        cp_qv.wait()

        def row(i, _):
            for db in range(D // 16):
                sl = pl.ds(db * 16, 16)
                z = kv[i, sl] + qvv[i, sl] + ev[i, sl]
                g = 1.0 / (1.0 + jnp.exp(-z))
                ev[i, sl] = g * qvv[i, pl.ds(D + db * 16, 16)]
            return 0
        lax.fori_loop(0, C, row, 0)

        pltpu.sync_copy(ev, aggr_sh.at[dstv], add=True)
        return 0

    lax.fori_loop(0, NCH, chunk, 0)
    plsc.subcore_barrier()

    for j in range(NR_TILE // ZR):
        r = row0 + j * ZR
        pltpu.sync_copy(aggr_sh.at[pl.ds(r, ZR)], kv)
        pltpu.sync_copy(kv, out_hbm.at[c, pl.ds(r, ZR)])


@functools.lru_cache(maxsize=None)
def _edge_kernel():
    mesh = plsc.VectorSubcoreMesh(
        core_axis_name="c", subcore_axis_name="s",
        num_cores=NC, num_subcores=NS)
    return functools.partial(
        pl.kernel,
        out_type=jax.ShapeDtypeStruct((NC, NPAD, D), jnp.float32),
        mesh=mesh,
        scratch_types=[
            pltpu.VMEM((C,), jnp.int32),
            pltpu.VMEM((C,), jnp.int32),
            pltpu.VMEM((C, D), jnp.float32),
            pltpu.VMEM((C, 2 * D), jnp.float32),
            pltpu.VMEM((C, D), jnp.float32),
            pltpu.VMEM_SHARED((NPAD, D), jnp.float32),
            pltpu.SemaphoreType.DMA,
            pltpu.SemaphoreType.DMA,
        ],
    )(_edge_body)


RN = 2000
GN = N // RN
RE = 8000
GE = E // RE


def _dense_b(x_ref, wk, bk, wqv, bqv, ws, bs, k_ref, qv_ref, s_ref):
    xb = x_ref[...]
    k_ref[...] = jnp.dot(xb, wk[...], preferred_element_type=jnp.float32) + bk[...]
    qv_ref[...] = jnp.dot(xb, wqv[...], preferred_element_type=jnp.float32) + bqv[...]
    s_ref[...] = jnp.dot(xb, ws[...], preferred_element_type=jnp.float32) + bs[...]


@functools.lru_cache(maxsize=None)
def _dense_kernel():
    full = lambda shape: pl.BlockSpec(shape, lambda i: (0, 0))
    return pl.pallas_call(
        _dense_b,
        grid=(GN,),
        in_specs=[
            pl.BlockSpec((RN, D), lambda i: (i, 0)),
            full((D, D)), full((1, D)),
            full((D, 2 * D)), full((1, 2 * D)),
            full((D, D)), full((1, D)),
        ],
        out_specs=[
            pl.BlockSpec((RN, D), lambda i: (i, 0)),
            pl.BlockSpec((RN, 2 * D), lambda i: (i, 0)),
            pl.BlockSpec((RN, D), lambda i: (i, 0)),
        ],
        out_shape=[
            jax.ShapeDtypeStruct((N, D), jnp.float32),
            jax.ShapeDtypeStruct((N, 2 * D), jnp.float32),
            jax.ShapeDtypeStruct((N, D), jnp.float32),
        ],
    )


def _edgeproj_b(xe_ref, w1, b1, w2, b2, e1_ref, e2_ref):
    xe = xe_ref[...]
    e1_ref[...] = jnp.dot(xe, w1[...], preferred_element_type=jnp.float32) + b1[...]
    e2_ref[...] = jnp.dot(xe, w2[...], preferred_element_type=jnp.float32) + b2[...]


@functools.lru_cache(maxsize=None)
def _edgeproj_kernel():
    full = lambda shape: pl.BlockSpec(shape, lambda i: (0, 0))
    return pl.pallas_call(
        _edgeproj_b,
        grid=(GE,),
        in_specs=[
            pl.BlockSpec((RE, DE), lambda i: (i, 0)),
            full((DE, D)), full((1, D)),
            full((DE, D)), full((1, D)),
        ],
        out_specs=[
            pl.BlockSpec((RE, D), lambda i: (i, 0)),
            pl.BlockSpec((RE, D), lambda i: (i, 0)),
        ],
        out_shape=[
            jax.ShapeDtypeStruct((E, D), jnp.float32),
            jax.ShapeDtypeStruct((E, D), jnp.float32),
        ],
    )


def _combine_b(aggr_ref, s_ref, b_ref, pre_ref, sum_ref, ssq_ref):
    i = pl.program_id(0)
    blk = aggr_ref[0] + aggr_ref[1] + s_ref[...] + b_ref[...]
    pre_ref[...] = blk

    @pl.when(i == 0)
    def _():
        sum_ref[...] = jnp.zeros_like(sum_ref)
        ssq_ref[...] = jnp.zeros_like(ssq_ref)

    sum_ref[...] += jnp.sum(blk, axis=0, keepdims=True)
    ssq_ref[...] += jnp.sum(blk * blk, axis=0, keepdims=True)


@functools.lru_cache(maxsize=None)
def _combine_kernel():
    full = lambda shape: pl.BlockSpec(shape, lambda i: tuple(0 for _ in shape))
    return pl.pallas_call(
        _combine_b,
        grid=(GN,),
        in_specs=[
            pl.BlockSpec((NC, RN, D), lambda i: (0, i, 0)),
            pl.BlockSpec((RN, D), lambda i: (i, 0)),
            full((1, D)),
        ],
        out_specs=[
            pl.BlockSpec((RN, D), lambda i: (i, 0)),
            full((1, D)),
            full((1, D)),
        ],
        out_shape=[
            jax.ShapeDtypeStruct((N, D), jnp.float32),
            jax.ShapeDtypeStruct((1, D), jnp.float32),
            jax.ShapeDtypeStruct((1, D), jnp.float32),
        ],
    )


def _bn_h(pre_ref, sum_ref, ssq_ref, g_ref, bb_ref):
    mu = sum_ref[...] / N
    var = ssq_ref[...] / N - mu * mu
    xn = (pre_ref[...] - mu) * lax.rsqrt(var + 1e-5)
    return jnp.maximum(xn * g_ref[...] + bb_ref[...], 0.0)


def _bn_dense_b(pre_ref, sum_ref, ssq_ref, g_ref, bb_ref,
                wk, bk, wqv, bqv, ws, bs, k_ref, qv_ref, s_ref):
    h = _bn_h(pre_ref, sum_ref, ssq_ref, g_ref, bb_ref)
    k_ref[...] = jnp.dot(h, wk[...], preferred_element_type=jnp.float32) + bk[...]
    qv_ref[...] = jnp.dot(h, wqv[...], preferred_element_type=jnp.float32) + bqv[...]
    s_ref[...] = jnp.dot(h, ws[...], preferred_element_type=jnp.float32) + bs[...]


@functools.lru_cache(maxsize=None)
def _bn_dense_kernel():
    full = lambda shape: pl.BlockSpec(shape, lambda i: (0, 0))
    return pl.pallas_call(
        _bn_dense_b,
        grid=(GN,),
        in_specs=[
            pl.BlockSpec((RN, D), lambda i: (i, 0)),
            full((1, D)), full((1, D)), full((1, D)), full((1, D)),
            full((D, D)), full((1, D)),
            full((D, 2 * D)), full((1, 2 * D)),
            full((D, D)), full((1, D)),
        ],
        out_specs=[
            pl.BlockSpec((RN, D), lambda i: (i, 0)),
            pl.BlockSpec((RN, 2 * D), lambda i: (i, 0)),
            pl.BlockSpec((RN, D), lambda i: (i, 0)),
        ],
        out_shape=[
            jax.ShapeDtypeStruct((N, D), jnp.float32),
            jax.ShapeDtypeStruct((N, 2 * D), jnp.float32),
            jax.ShapeDtypeStruct((N, D), jnp.float32),
        ],
    )


def _bn_apply_b(pre_ref, sum_ref, ssq_ref, g_ref, bb_ref, out_ref):
    out_ref[...] = _bn_h(pre_ref, sum_ref, ssq_ref, g_ref, bb_ref)


@functools.lru_cache(maxsize=None)
def _bn_apply_kernel():
    full = lambda shape: pl.BlockSpec(shape, lambda i: (0, 0))
    return pl.pallas_call(
        _bn_apply_b,
        grid=(GN,),
        in_specs=[
            pl.BlockSpec((RN, D), lambda i: (i, 0)),
            full((1, D)), full((1, D)), full((1, D)), full((1, D)),
        ],
        out_specs=pl.BlockSpec((RN, D), lambda i: (i, 0)),
        out_shape=jax.ShapeDtypeStruct((N, D), jnp.float32),
    )


def kernel(x, edge_index, x_edge,
           Wk1, bk1, Wq1, bq1, Wv1, bv1, We1, be1, Ws1, b1, bng1, bnb1,
           Wk2, bk2, Wq2, bq2, Wv2, bv2, We2, be2, Ws2, b2, bng2, bnb2):
    src = edge_index[0]
    dst = edge_index[1]

    r1 = lambda v: v.reshape(1, D)
    wqv1 = jnp.concatenate([Wq1.T, Wv1.T], axis=1)
    bqv1 = jnp.concatenate([bq1, bv1]).reshape(1, 2 * D)
    wqv2 = jnp.concatenate([Wq2.T, Wv2.T], axis=1)
    bqv2 = jnp.concatenate([bq2, bv2]).reshape(1, 2 * D)

    e1, e2 = _edgeproj_kernel()(x_edge, We1.T, r1(be1), We2.T, r1(be2))

    k1, qv1, s1 = _dense_kernel()(
        x, Wk1.T, r1(bk1), wqv1, bqv1, Ws1.T, r1(b1))
    aggr1 = _edge_kernel()(src, dst, k1, qv1, e1)
    pre1, sum1, ssq1 = _combine_kernel()(aggr1, s1, jnp.zeros((1, D), jnp.float32))

    k2, qv2, s2 = _bn_dense_kernel()(
        pre1, sum1, ssq1, r1(bng1), r1(bnb1),
        Wk2.T, r1(bk2), wqv2, bqv2, Ws2.T, r1(b2))
    aggr2 = _edge_kernel()(src, dst, k2, qv2, e2)
    pre2, sum2, ssq2 = _combine_kernel()(aggr2, s2, jnp.zeros((1, D), jnp.float32))

    return _bn_apply_kernel()(pre2, sum2, ssq2, r1(bng2), r1(bnb2))


# speedup vs baseline: 3.5921x; 2.7418x over previous
import functools

import jax
import jax.numpy as jnp
from jax import lax
from jax.experimental import pallas as pl
from jax.experimental.pallas import tpu as pltpu
from jax.experimental.pallas import tpu_sc as plsc

N = 10000
E = 320000
D = 128
DE = 16

NC = 2
NS = 16
NW = NC * NS

EW = E // NW
C = 40
NCH = EW // C
NPAD = 10240
NR_TILE = NPAD // NS


def _edge_body(src_hbm, dst_hbm, k_hbm, qv_hbm, e_hbm, out_hbm,
               src0, src1, dst0, dst1, kv0, kv1, qv0, qv1, ev0, ev1,
               aggr_sh, isem0, isem1, dsem0, dsem1):
    c = lax.axis_index("c")
    s = lax.axis_index("s")
    w = c * NS + s

    srcb = (src0, src1)
    dstb = (dst0, dst1)
    kvb = (kv0, kv1)
    qvb = (qv0, qv1)
    evb = (ev0, ev1)
    isem = (isem0, isem1)
    dsem = (dsem0, dsem1)

    def zrow(i, _):
        for db in range(D // 16):
            kv0[i, pl.ds(db * 16, 16)] = jnp.zeros((16,), jnp.float32)
        return 0
    lax.fori_loop(0, C, zrow, 0)
    row0 = s * NR_TILE
    for j in range(NR_TILE // C):
        pltpu.sync_copy(kv0, aggr_sh.at[pl.ds(row0 + j * C, C)])
    plsc.subcore_barrier()

    base0 = w * EW

    def idx_load(t, b):
        base = base0 + t * C
        pltpu.async_copy(src_hbm.at[pl.ds(base, C)], srcb[b], isem[b])
        pltpu.async_copy(dst_hbm.at[pl.ds(base, C)], dstb[b], isem[b])

    def idx_wait(b):
        pltpu.make_async_copy(src_hbm.at[pl.ds(0, C)], srcb[b], isem[b]).wait()
        pltpu.make_async_copy(dst_hbm.at[pl.ds(0, C)], dstb[b], isem[b]).wait()

    def data_load(t, b):
        base = base0 + t * C
        pltpu.async_copy(k_hbm.at[dstb[b]], kvb[b], dsem[b])
        pltpu.async_copy(qv_hbm.at[srcb[b]], qvb[b], dsem[b])
        pltpu.async_copy(e_hbm.at[pl.ds(base, C)], evb[b], dsem[b])

    def data_wait(b):
        pltpu.make_async_copy(k_hbm.at[pl.ds(0, C)], kvb[b], dsem[b]).wait()
        pltpu.make_async_copy(qv_hbm.at[pl.ds(0, C)], qvb[b], dsem[b]).wait()
        pltpu.make_async_copy(e_hbm.at[pl.ds(0, C)], evb[b], dsem[b]).wait()

    def phase(t, b, nb):
        data_wait(b)

        @plsc.parallel_loop(0, C, 1, unroll=2)
        def _(i):
            for db in range(D // 16):
                sl = pl.ds(db * 16, 16)
                z = kvb[b][i, sl] + qvb[b][i, sl] + evb[b][i, sl]
                g = 1.0 / (1.0 + jnp.exp(-z))
                evb[b][i, sl] = g * qvb[b][i, pl.ds(D + db * 16, 16)]

        pltpu.sync_copy(evb[b], aggr_sh.at[dstb[b]], add=True)
        idx_load(jnp.minimum(t + 2, NCH - 1), b)
        idx_wait(nb)
        data_load(jnp.minimum(t + 1, NCH - 1), nb)

    idx_load(0, 0)
    idx_load(1, 1)
    idx_wait(0)
    data_load(0, 0)

    def giter(g, _):
        t = g * 2
        phase(t, 0, 1)
        phase(t + 1, 1, 0)
        return 0
    lax.fori_loop(0, NCH // 2, giter, 0)

    idx_wait(1)
    data_wait(0)
    plsc.subcore_barrier()

    for j in range(NR_TILE // C):
        r = row0 + j * C
        pltpu.sync_copy(aggr_sh.at[pl.ds(r, C)], kv0)
        pltpu.sync_copy(kv0, out_hbm.at[c, pl.ds(r, C)])


@functools.lru_cache(maxsize=None)
def _edge_kernel():
    mesh = plsc.VectorSubcoreMesh(
        core_axis_name="c", subcore_axis_name="s",
        num_cores=NC, num_subcores=NS)
    return functools.partial(
        pl.kernel,
        out_type=jax.ShapeDtypeStruct((NC, NPAD, D), jnp.float32),
        mesh=mesh,
        scratch_types=[
            pltpu.VMEM((C,), jnp.int32),
            pltpu.VMEM((C,), jnp.int32),
            pltpu.VMEM((C,), jnp.int32),
            pltpu.VMEM((C,), jnp.int32),
            pltpu.VMEM((C, D), jnp.float32),
            pltpu.VMEM((C, D), jnp.float32),
            pltpu.VMEM((C, 2 * D), jnp.float32),
            pltpu.VMEM((C, 2 * D), jnp.float32),
            pltpu.VMEM((C, D), jnp.float32),
            pltpu.VMEM((C, D), jnp.float32),
            pltpu.VMEM_SHARED((NPAD, D), jnp.float32),
            pltpu.SemaphoreType.DMA,
            pltpu.SemaphoreType.DMA,
            pltpu.SemaphoreType.DMA,
            pltpu.SemaphoreType.DMA,
        ],
    )(_edge_body)


RN = 2000
GN = N // RN
RE = 8000
GE = E // RE


def _dense_b(x_ref, wk, bk, wqv, bqv, ws, bs, k_ref, qv_ref, s_ref):
    xb = x_ref[...]
    k_ref[...] = jnp.dot(xb, wk[...], preferred_element_type=jnp.float32) + bk[...]
    qv_ref[...] = jnp.dot(xb, wqv[...], preferred_element_type=jnp.float32) + bqv[...]
    s_ref[...] = jnp.dot(xb, ws[...], preferred_element_type=jnp.float32) + bs[...]


@functools.lru_cache(maxsize=None)
def _dense_kernel():
    full = lambda shape: pl.BlockSpec(shape, lambda i: (0, 0))
    return pl.pallas_call(
        _dense_b,
        grid=(GN,),
        in_specs=[
            pl.BlockSpec((RN, D), lambda i: (i, 0)),
            full((D, D)), full((1, D)),
            full((D, 2 * D)), full((1, 2 * D)),
            full((D, D)), full((1, D)),
        ],
        out_specs=[
            pl.BlockSpec((RN, D), lambda i: (i, 0)),
            pl.BlockSpec((RN, 2 * D), lambda i: (i, 0)),
            pl.BlockSpec((RN, D), lambda i: (i, 0)),
        ],
        out_shape=[
            jax.ShapeDtypeStruct((N, D), jnp.float32),
            jax.ShapeDtypeStruct((N, 2 * D), jnp.float32),
            jax.ShapeDtypeStruct((N, D), jnp.float32),
        ],
    )


def _edgeproj_b(xe_ref, w1, b1, w2, b2, e1_ref, e2_ref):
    xe = xe_ref[...]
    e1_ref[...] = jnp.dot(xe, w1[...], preferred_element_type=jnp.float32) + b1[...]
    e2_ref[...] = jnp.dot(xe, w2[...], preferred_element_type=jnp.float32) + b2[...]


@functools.lru_cache(maxsize=None)
def _edgeproj_kernel():
    full = lambda shape: pl.BlockSpec(shape, lambda i: (0, 0))
    return pl.pallas_call(
        _edgeproj_b,
        grid=(GE,),
        in_specs=[
            pl.BlockSpec((RE, DE), lambda i: (i, 0)),
            full((DE, D)), full((1, D)),
            full((DE, D)), full((1, D)),
        ],
        out_specs=[
            pl.BlockSpec((RE, D), lambda i: (i, 0)),
            pl.BlockSpec((RE, D), lambda i: (i, 0)),
        ],
        out_shape=[
            jax.ShapeDtypeStruct((E, D), jnp.float32),
            jax.ShapeDtypeStruct((E, D), jnp.float32),
        ],
    )


def _combine_b(aggr_ref, s_ref, b_ref, pre_ref, sum_ref, ssq_ref):
    i = pl.program_id(0)
    blk = aggr_ref[0] + aggr_ref[1] + s_ref[...] + b_ref[...]
    pre_ref[...] = blk

    @pl.when(i == 0)
    def _():
        sum_ref[...] = jnp.zeros_like(sum_ref)
        ssq_ref[...] = jnp.zeros_like(ssq_ref)

    sum_ref[...] += jnp.sum(blk, axis=0, keepdims=True)
    ssq_ref[...] += jnp.sum(blk * blk, axis=0, keepdims=True)


@functools.lru_cache(maxsize=None)
def _combine_kernel():
    full = lambda shape: pl.BlockSpec(shape, lambda i: tuple(0 for _ in shape))
    return pl.pallas_call(
        _combine_b,
        grid=(GN,),
        in_specs=[
            pl.BlockSpec((NC, RN, D), lambda i: (0, i, 0)),
            pl.BlockSpec((RN, D), lambda i: (i, 0)),
            full((1, D)),
        ],
        out_specs=[
            pl.BlockSpec((RN, D), lambda i: (i, 0)),
            full((1, D)),
            full((1, D)),
        ],
        out_shape=[
            jax.ShapeDtypeStruct((N, D), jnp.float32),
            jax.ShapeDtypeStruct((1, D), jnp.float32),
            jax.ShapeDtypeStruct((1, D), jnp.float32),
        ],
    )


def _bn_h(pre_ref, sum_ref, ssq_ref, g_ref, bb_ref):
    mu = sum_ref[...] / N
    var = ssq_ref[...] / N - mu * mu
    xn = (pre_ref[...] - mu) * lax.rsqrt(var + 1e-5)
    return jnp.maximum(xn * g_ref[...] + bb_ref[...], 0.0)


def _bn_dense_b(pre_ref, sum_ref, ssq_ref, g_ref, bb_ref,
                wk, bk, wqv, bqv, ws, bs, k_ref, qv_ref, s_ref):
    h = _bn_h(pre_ref, sum_ref, ssq_ref, g_ref, bb_ref)
    k_ref[...] = jnp.dot(h, wk[...], preferred_element_type=jnp.float32) + bk[...]
    qv_ref[...] = jnp.dot(h, wqv[...], preferred_element_type=jnp.float32) + bqv[...]
    s_ref[...] = jnp.dot(h, ws[...], preferred_element_type=jnp.float32) + bs[...]


@functools.lru_cache(maxsize=None)
def _bn_dense_kernel():
    full = lambda shape: pl.BlockSpec(shape, lambda i: (0, 0))
    return pl.pallas_call(
        _bn_dense_b,
        grid=(GN,),
        in_specs=[
            pl.BlockSpec((RN, D), lambda i: (i, 0)),
            full((1, D)), full((1, D)), full((1, D)), full((1, D)),
            full((D, D)), full((1, D)),
            full((D, 2 * D)), full((1, 2 * D)),
            full((D, D)), full((1, D)),
        ],
        out_specs=[
            pl.BlockSpec((RN, D), lambda i: (i, 0)),
            pl.BlockSpec((RN, 2 * D), lambda i: (i, 0)),
            pl.BlockSpec((RN, D), lambda i: (i, 0)),
        ],
        out_shape=[
            jax.ShapeDtypeStruct((N, D), jnp.float32),
            jax.ShapeDtypeStruct((N, 2 * D), jnp.float32),
            jax.ShapeDtypeStruct((N, D), jnp.float32),
        ],
    )


def _bn_apply_b(pre_ref, sum_ref, ssq_ref, g_ref, bb_ref, out_ref):
    out_ref[...] = _bn_h(pre_ref, sum_ref, ssq_ref, g_ref, bb_ref)


@functools.lru_cache(maxsize=None)
def _bn_apply_kernel():
    full = lambda shape: pl.BlockSpec(shape, lambda i: (0, 0))
    return pl.pallas_call(
        _bn_apply_b,
        grid=(GN,),
        in_specs=[
            pl.BlockSpec((RN, D), lambda i: (i, 0)),
            full((1, D)), full((1, D)), full((1, D)), full((1, D)),
        ],
        out_specs=pl.BlockSpec((RN, D), lambda i: (i, 0)),
        out_shape=jax.ShapeDtypeStruct((N, D), jnp.float32),
    )


def kernel(x, edge_index, x_edge,
           Wk1, bk1, Wq1, bq1, Wv1, bv1, We1, be1, Ws1, b1, bng1, bnb1,
           Wk2, bk2, Wq2, bq2, Wv2, bv2, We2, be2, Ws2, b2, bng2, bnb2):
    src = edge_index[0]
    dst = edge_index[1]

    r1 = lambda v: v.reshape(1, D)
    wqv1 = jnp.concatenate([Wq1.T, Wv1.T], axis=1)
    bqv1 = jnp.concatenate([bq1, bv1]).reshape(1, 2 * D)
    wqv2 = jnp.concatenate([Wq2.T, Wv2.T], axis=1)
    bqv2 = jnp.concatenate([bq2, bv2]).reshape(1, 2 * D)

    e1, e2 = _edgeproj_kernel()(x_edge, We1.T, r1(be1), We2.T, r1(be2))

    k1, qv1, s1 = _dense_kernel()(
        x, Wk1.T, r1(bk1), wqv1, bqv1, Ws1.T, r1(b1))
    aggr1 = _edge_kernel()(src, dst, k1, qv1, e1)
    pre1, sum1, ssq1 = _combine_kernel()(aggr1, s1, jnp.zeros((1, D), jnp.float32))

    k2, qv2, s2 = _bn_dense_kernel()(
        pre1, sum1, ssq1, r1(bng1), r1(bnb1),
        Wk2.T, r1(bk2), wqv2, bqv2, Ws2.T, r1(b2))
    aggr2 = _edge_kernel()(src, dst, k2, qv2, e2)
    pre2, sum2, ssq2 = _combine_kernel()(aggr2, s2, jnp.zeros((1, D), jnp.float32))

    return _bn_apply_kernel()(pre2, sum2, ssq2, r1(bng2), r1(bnb2))
